# Initial kernel scaffold; baseline (speedup 1.0000x reference)
import jax
import jax.numpy as jnp
from jax.experimental import pallas as pl


def kernel(x_encounter, x_diagnosis, x_medication, edge_has_dx, edge_dx_of, edge_has_med, edge_med_of, W_enc, b_enc, Emb_dx, Emb_med, W_rel1, W_root1, b1, W_rel2, W_root2, b2, W_cls, b_cls):
    raise NotImplementedError("write your pallas kernel here")


# speedup vs baseline: 10.0549x; 10.0549x over previous
import functools

import jax
import jax.numpy as jnp
from jax import lax
from jax.experimental import pallas as pl
from jax.experimental.pallas import tpu as pltpu
from jax.experimental.pallas import tpu_sc as plsc

N_ENC = 20000
N_DX = 20000
N_MED = 10000
H = 128
E = 150000
NC = 2
NS = 16
CHUNK = 128

TCH = -(-E // (NS * CHUNK))
EP = NS * TCH * CHUNK
DST_PAD = 0x3FFFFFFF


def _geom(n_dst):
  dh = n_dst // NC
  srows = ((dh + 1 + 15) // 16) * 16
  rps = ((-(-srows // NS) + 7) // 8) * 8
  zc = -(-rps // CHUNK)
  wps = ((-(-dh // NS) + 7) // 8) * 8
  return dh, srows, rps, zc, wps


def _mm_body(x_ref, w_ref, b_ref, o_ref):
  o_ref[...] = (
      jnp.dot(x_ref[...], w_ref[...], preferred_element_type=jnp.float32)
      + b_ref[...]
  )


def _mm(x, w, b, bm=400):
  m, k = x.shape
  n = w.shape[1]
  return pl.pallas_call(
      _mm_body,
      grid=(m // bm,),
      in_specs=[
          pl.BlockSpec((bm, k), lambda i: (i, 0)),
          pl.BlockSpec((k, n), lambda i: (0, 0)),
          pl.BlockSpec((1, n), lambda i: (0, 0)),
      ],
      out_specs=pl.BlockSpec((bm, n), lambda i: (i, 0)),
      out_shape=jax.ShapeDtypeStruct((m, n), jnp.float32),
  )(x, w, b.reshape(1, n))


def _comb1_body(root_ref, a_ref, c_ref, o_ref):
  inv = 1.0 / jnp.maximum(c_ref[:, :1], 1.0)
  o_ref[...] = jnp.maximum(root_ref[...] + a_ref[...] * inv, 0.0)


def _comb1(root, agg, cnt, bm=400):
  m = root.shape[0]
  return pl.pallas_call(
      _comb1_body,
      grid=(m // bm,),
      in_specs=[
          pl.BlockSpec((bm, H), lambda i: (i, 0)),
          pl.BlockSpec((bm, H), lambda i: (i, 0)),
          pl.BlockSpec((bm, H), lambda i: (i, 0)),
      ],
      out_specs=pl.BlockSpec((bm, H), lambda i: (i, 0)),
      out_shape=jax.ShapeDtypeStruct((m, H), jnp.float32),
  )(root, agg, cnt)


def _comb2_body(root_ref, a1_ref, c1_ref, a3_ref, c3_ref, o_ref):
  inv1 = 1.0 / jnp.maximum(c1_ref[:, :1], 1.0)
  inv3 = 1.0 / jnp.maximum(c3_ref[:, :1], 1.0)
  o_ref[...] = jnp.maximum(
      root_ref[...] + a1_ref[...] * inv1 + a3_ref[...] * inv3, 0.0)


def _comb2(root, a1, c1, a3, c3, bm=400):
  m = root.shape[0]
  return pl.pallas_call(
      _comb2_body,
      grid=(m // bm,),
      in_specs=[pl.BlockSpec((bm, H), lambda i: (i, 0))] * 5,
      out_specs=pl.BlockSpec((bm, H), lambda i: (i, 0)),
      out_shape=jax.ShapeDtypeStruct((m, H), jnp.float32),
  )(root, a1, c1, a3, c3)


def _final_body(root_ref, a1_ref, c1_ref, a3_ref, c3_ref, wc_ref, bc_ref,
                o_ref):
  inv1 = 1.0 / jnp.maximum(c1_ref[:, :1], 1.0)
  inv3 = 1.0 / jnp.maximum(c3_ref[:, :1], 1.0)
  h = jnp.maximum(root_ref[...] + a1_ref[...] * inv1 + a3_ref[...] * inv3,
                  0.0)
  o_ref[...] = jnp.sum(h * wc_ref[...], axis=1, keepdims=True) + bc_ref[...]


def _final(root, a1, c1, a3, c3, w_cls, b_cls, bm=400):
  m = root.shape[0]
  return pl.pallas_call(
      _final_body,
      grid=(m // bm,),
      in_specs=[pl.BlockSpec((bm, H), lambda i: (i, 0))] * 5 + [
          pl.BlockSpec((1, H), lambda i: (0, 0)),
          pl.BlockSpec((1, 1), lambda i: (0, 0)),
      ],
      out_specs=pl.BlockSpec((bm, 1), lambda i: (i, 0)),
      out_shape=jax.ShapeDtypeStruct((m, 1), jnp.float32),
  )(root, a1, c1, a3, c3, w_cls.reshape(1, H), b_cls.reshape(1, 1))


def _gather_rows(table, idx, n_rows):
  nfull = n_rows // CHUNK
  n_chunks = nfull + (1 if n_rows % CHUNK else 0)
  t_per_w = -(-n_chunks // (NC * NS))

  def body(table_ref, idx_ref, out_ref, idxv, rows, sem):
    c = lax.axis_index("c")
    s = lax.axis_index("s")
    wid = s * NC + c

    def step(t, carry):
      cid = wid * t_per_w + t

      @pl.when(cid < n_chunks)
      def _():
        off = jnp.minimum(cid * CHUNK, n_rows - CHUNK)
        pltpu.sync_copy(idx_ref.at[pl.ds(off, CHUNK)], idxv)
        pltpu.async_copy(table_ref.at[idxv], rows, sem).wait()
        pltpu.sync_copy(rows, out_ref.at[pl.ds(off, CHUNK)])

      return carry

    lax.fori_loop(0, t_per_w, step, 0)

  fn = pl.kernel(
      body,
      out_type=jax.ShapeDtypeStruct((n_rows, H), jnp.float32),
      mesh=plsc.VectorSubcoreMesh(core_axis_name="c", subcore_axis_name="s"),
      scratch_types=[
          pltpu.VMEM((CHUNK,), jnp.int32),
          pltpu.VMEM((CHUNK, H), jnp.float32),
          pltpu.SemaphoreType.DMA,
      ],
  )
  return fn(table, idx)


def _seg_agg(table, src, dst, n_dst):
  dh, srows, rps, zc, wps = _geom(n_dst)

  def body(table_ref, src_ref, dst_ref, agg_ref, srcv, dstv, didx, rows,
           zerob, aggS, sem):
    c = lax.axis_index("c")
    s = lax.axis_index("s")
    lo = c * dh

    zv = jnp.zeros((16,), jnp.float32)

    def zrow(r, carry):
      for g in range(H // 16):
        zerob[r, pl.ds(g * 16, 16)] = zv
      return carry

    lax.fori_loop(0, CHUNK, zrow, 0)

    def zcopy(k, carry):
      off = jnp.minimum(s * rps + k * CHUNK, srows - CHUNK)
      pltpu.sync_copy(zerob, aggS.at[pl.ds(off, CHUNK)])
      return carry

    lax.fori_loop(0, zc, zcopy, 0)
    plsc.subcore_barrier()

    def step(t, carry):
      off = (s * TCH + t) * CHUNK
      pltpu.sync_copy(src_ref.at[pl.ds(off, CHUNK)], srcv)
      pltpu.sync_copy(dst_ref.at[pl.ds(off, CHUNK)], dstv)
      pltpu.async_copy(table_ref.at[srcv], rows, sem).wait()
      for g in range(CHUNK // 16):
        d = dstv[pl.ds(g * 16, 16)]
        li = d - lo
        valid = (d >= lo) & (li < dh)
        didx[0, pl.ds(g * 16, 16)] = jnp.where(valid, li, dh)
      pltpu.sync_copy(rows, aggS.at[didx.at[0]], add=True)
      return carry

    lax.fori_loop(0, TCH, step, 0)
    plsc.subcore_barrier()

    start = jnp.minimum(s * wps, dh - wps)
    pltpu.sync_copy(aggS.at[pl.ds(start, wps)],
                    agg_ref.at[pl.ds(lo + start, wps)])

  fn = pl.kernel(
      body,
      out_type=jax.ShapeDtypeStruct((n_dst, H), jnp.float32),
      mesh=plsc.VectorSubcoreMesh(core_axis_name="c", subcore_axis_name="s"),
      scratch_types=(
          pltpu.VMEM((CHUNK,), jnp.int32),
          pltpu.VMEM((CHUNK,), jnp.int32),
          pltpu.VMEM((1, CHUNK), jnp.int32),
          pltpu.VMEM((CHUNK, H), jnp.float32),
          pltpu.VMEM((CHUNK, H), jnp.float32),
          pltpu.VMEM_SHARED((srows, H), jnp.float32),
          pltpu.SemaphoreType.DMA,
      ),
  )
  return fn(table, src, dst)


def _edge_counts(d0, d1, d2, d3):
  dsts = (N_DX, N_ENC, N_MED, N_ENC)
  srows_max = max(_geom(n)[1] for n in dsts)

  def body(d0_ref, d1_ref, d2_ref, d3_ref, c0_ref, c1_ref, c2_ref, c3_ref,
           dstv, didx, onesb, zerob, cntS):
    c = lax.axis_index("c")
    s = lax.axis_index("s")

    zv = jnp.zeros((16,), jnp.float32)
    ov = jnp.ones((16,), jnp.float32)

    def frow(r, carry):
      for g in range(H // 16):
        zerob[r, pl.ds(g * 16, 16)] = zv
        onesb[r, pl.ds(g * 16, 16)] = ov
      return carry

    lax.fori_loop(0, CHUNK, frow, 0)

    for dst_ref, cnt_ref, n_dst in zip(
        (d0_ref, d1_ref, d2_ref, d3_ref),
        (c0_ref, c1_ref, c2_ref, c3_ref), dsts):
      dh, srows, rps, zc, wps = _geom(n_dst)
      lo = c * dh

      def zcopy(k, carry):
        off = jnp.minimum(s * rps + k * CHUNK, srows - CHUNK)
        pltpu.sync_copy(zerob, cntS.at[pl.ds(off, CHUNK)])
        return carry

      lax.fori_loop(0, zc, zcopy, 0)
      plsc.subcore_barrier()

      def step(t, carry):
        off = (s * TCH + t) * CHUNK
        pltpu.sync_copy(dst_ref.at[pl.ds(off, CHUNK)], dstv)
        for g in range(CHUNK // 16):
          d = dstv[pl.ds(g * 16, 16)]
          li = d - lo
          valid = (d >= lo) & (li < dh)
          didx[0, pl.ds(g * 16, 16)] = jnp.where(valid, li, dh)
        pltpu.sync_copy(onesb, cntS.at[didx.at[0]], add=True)
        return carry

      lax.fori_loop(0, TCH, step, 0)
      plsc.subcore_barrier()

      start = jnp.minimum(s * wps, dh - wps)
      pltpu.sync_copy(cntS.at[pl.ds(start, wps)],
                      cnt_ref.at[pl.ds(lo + start, wps)])
      plsc.subcore_barrier()

  fn = pl.kernel(
      body,
      out_type=tuple(
          jax.ShapeDtypeStruct((n, H), jnp.float32) for n in dsts),
      mesh=plsc.VectorSubcoreMesh(core_axis_name="c", subcore_axis_name="s"),
      scratch_types=(
          pltpu.VMEM((CHUNK,), jnp.int32),
          pltpu.VMEM((1, CHUNK), jnp.int32),
          pltpu.VMEM((CHUNK, H), jnp.float32),
          pltpu.VMEM((CHUNK, H), jnp.float32),
          pltpu.VMEM_SHARED((srows_max, H), jnp.float32),
      ),
  )
  return fn(d0, d1, d2, d3)


def _pad_edges(e):
  src = e[0].astype(jnp.int32)
  dst = e[1].astype(jnp.int32)
  pad = EP - E
  src = jnp.concatenate([src, jnp.zeros((pad,), jnp.int32)])
  dst = jnp.concatenate([dst, jnp.full((pad,), DST_PAD, jnp.int32)])
  return src, dst


def kernel(x_encounter, x_diagnosis, x_medication, edge_has_dx, edge_dx_of,
           edge_has_med, edge_med_of, W_enc, b_enc, Emb_dx, Emb_med, W_rel1,
           W_root1, b1, W_rel2, W_root2, b2, W_cls, b_cls):
  zb = jnp.zeros((H,), jnp.float32)

  h0_enc = _mm(x_encounter, W_enc, b_enc)
  h0_dx = _gather_rows(Emb_dx, x_diagnosis.astype(jnp.int32), N_DX)
  h0_med = _gather_rows(Emb_med, x_medication.astype(jnp.int32), N_MED)

  s0, d0 = _pad_edges(edge_has_dx)
  s1, d1 = _pad_edges(edge_dx_of)
  s2, d2 = _pad_edges(edge_has_med)
  s3, d3 = _pad_edges(edge_med_of)

  c0, c1, c2, c3 = _edge_counts(d0, d1, d2, d3)

  m0 = _mm(h0_enc, W_rel1[0], zb)
  m1 = _mm(h0_dx, W_rel1[1], zb)
  m2 = _mm(h0_enc, W_rel1[2], zb)
  m3 = _mm(h0_med, W_rel1[3], zb)
  r_enc = _mm(h0_enc, W_root1, b1)
  r_dx = _mm(h0_dx, W_root1, b1)
  r_med = _mm(h0_med, W_root1, b1)

  a0 = _seg_agg(m0, s0, d0, N_DX)
  a1 = _seg_agg(m1, s1, d1, N_ENC)
  a2 = _seg_agg(m2, s2, d2, N_MED)
  a3 = _seg_agg(m3, s3, d3, N_ENC)

  h1_dx = _comb1(r_dx, a0, c0)
  h1_med = _comb1(r_med, a2, c2)
  h1_enc = _comb2(r_enc, a1, c1, a3, c3)

  m1b = _mm(h1_dx, W_rel2[1], zb)
  m3b = _mm(h1_med, W_rel2[3], zb)
  r2 = _mm(h1_enc, W_root2, b2)

  a1b = _seg_agg(m1b, s1, d1, N_ENC)
  a3b = _seg_agg(m3b, s3, d3, N_ENC)

  logits = _final(r2, a1b, c1, a3b, c3, W_cls, b_cls)
  return logits.reshape(-1)


# speedup vs baseline: 12.3147x; 1.2247x over previous
import functools

import jax
import jax.numpy as jnp
from jax import lax
from jax.experimental import pallas as pl
from jax.experimental.pallas import tpu as pltpu
from jax.experimental.pallas import tpu_sc as plsc

N_ENC = 20000
N_DX = 20000
N_MED = 10000
H = 128
E = 150000
NC = 2
NS = 16
CHUNK = 128

TCH = -(-E // (NS * CHUNK))
EP = NS * TCH * CHUNK
DST_PAD = 0x3FFFFFFF


def _geom(n_dst):
  dh = n_dst // NC
  srows = ((dh + 1 + 15) // 16) * 16
  rps = ((-(-srows // NS) + 7) // 8) * 8
  zc = -(-rps // CHUNK)
  wps = ((-(-dh // NS) + 7) // 8) * 8
  return dh, srows, rps, zc, wps


def _mm_body(x_ref, w_ref, b_ref, o_ref):
  o_ref[...] = (
      jnp.dot(x_ref[...], w_ref[...], preferred_element_type=jnp.float32)
      + b_ref[...]
  )


def _mm(x, w, b, bm=400):
  m, k = x.shape
  n = w.shape[1]
  return pl.pallas_call(
      _mm_body,
      grid=(m // bm,),
      in_specs=[
          pl.BlockSpec((bm, k), lambda i: (i, 0)),
          pl.BlockSpec((k, n), lambda i: (0, 0)),
          pl.BlockSpec((1, n), lambda i: (0, 0)),
      ],
      out_specs=pl.BlockSpec((bm, n), lambda i: (i, 0)),
      out_shape=jax.ShapeDtypeStruct((m, n), jnp.float32),
  )(x, w, b.reshape(1, n))


def _comb1_body(root_ref, a_ref, c_ref, o_ref):
  inv = 1.0 / jnp.maximum(c_ref[:, :1], 1.0)
  o_ref[...] = jnp.maximum(root_ref[...] + a_ref[...] * inv, 0.0)


def _comb1(root, agg, cnt, bm=400):
  m = root.shape[0]
  return pl.pallas_call(
      _comb1_body,
      grid=(m // bm,),
      in_specs=[
          pl.BlockSpec((bm, H), lambda i: (i, 0)),
          pl.BlockSpec((bm, H), lambda i: (i, 0)),
          pl.BlockSpec((bm, H), lambda i: (i, 0)),
      ],
      out_specs=pl.BlockSpec((bm, H), lambda i: (i, 0)),
      out_shape=jax.ShapeDtypeStruct((m, H), jnp.float32),
  )(root, agg, cnt)


def _comb2_body(root_ref, a1_ref, c1_ref, a3_ref, c3_ref, o_ref):
  inv1 = 1.0 / jnp.maximum(c1_ref[:, :1], 1.0)
  inv3 = 1.0 / jnp.maximum(c3_ref[:, :1], 1.0)
  o_ref[...] = jnp.maximum(
      root_ref[...] + a1_ref[...] * inv1 + a3_ref[...] * inv3, 0.0)


def _comb2(root, a1, c1, a3, c3, bm=400):
  m = root.shape[0]
  return pl.pallas_call(
      _comb2_body,
      grid=(m // bm,),
      in_specs=[pl.BlockSpec((bm, H), lambda i: (i, 0))] * 5,
      out_specs=pl.BlockSpec((bm, H), lambda i: (i, 0)),
      out_shape=jax.ShapeDtypeStruct((m, H), jnp.float32),
  )(root, a1, c1, a3, c3)


def _final_body(root_ref, a1_ref, c1_ref, a3_ref, c3_ref, wc_ref, bc_ref,
                o_ref):
  inv1 = 1.0 / jnp.maximum(c1_ref[:, :1], 1.0)
  inv3 = 1.0 / jnp.maximum(c3_ref[:, :1], 1.0)
  h = jnp.maximum(root_ref[...] + a1_ref[...] * inv1 + a3_ref[...] * inv3,
                  0.0)
  o_ref[...] = jnp.sum(h * wc_ref[...], axis=1, keepdims=True) + bc_ref[...]


def _final(root, a1, c1, a3, c3, w_cls, b_cls, bm=400):
  m = root.shape[0]
  return pl.pallas_call(
      _final_body,
      grid=(m // bm,),
      in_specs=[pl.BlockSpec((bm, H), lambda i: (i, 0))] * 5 + [
          pl.BlockSpec((1, H), lambda i: (0, 0)),
          pl.BlockSpec((1, 1), lambda i: (0, 0)),
      ],
      out_specs=pl.BlockSpec((bm, 1), lambda i: (i, 0)),
      out_shape=jax.ShapeDtypeStruct((m, 1), jnp.float32),
  )(root, a1, c1, a3, c3, w_cls.reshape(1, H), b_cls.reshape(1, 1))


def _gather_rows(table, idx, n_rows):
  nfull = n_rows // CHUNK
  n_chunks = nfull + (1 if n_rows % CHUNK else 0)
  t_per_w = -(-n_chunks // (NC * NS))

  def body(table_ref, idx_ref, out_ref, idxv, rows, sem):
    c = lax.axis_index("c")
    s = lax.axis_index("s")
    wid = s * NC + c

    def step(t, carry):
      cid = wid * t_per_w + t

      @pl.when(cid < n_chunks)
      def _():
        off = jnp.minimum(cid * CHUNK, n_rows - CHUNK)
        pltpu.sync_copy(idx_ref.at[pl.ds(off, CHUNK)], idxv)
        pltpu.async_copy(table_ref.at[idxv], rows, sem).wait()
        pltpu.sync_copy(rows, out_ref.at[pl.ds(off, CHUNK)])

      return carry

    lax.fori_loop(0, t_per_w, step, 0)

  fn = pl.kernel(
      body,
      out_type=jax.ShapeDtypeStruct((n_rows, H), jnp.float32),
      mesh=plsc.VectorSubcoreMesh(core_axis_name="c", subcore_axis_name="s"),
      scratch_types=[
          pltpu.VMEM((CHUNK,), jnp.int32),
          pltpu.VMEM((CHUNK, H), jnp.float32),
          pltpu.SemaphoreType.DMA,
      ],
  )
  return fn(table, idx)


def _seg_agg(table, src, dst, n_dst):
  dh, srows, rps, zc, wps = _geom(n_dst)
  epw = TCH * CHUNK

  def body(table_ref, src_ref, dst_ref, agg_ref, sidxA, sidxB, didxA, didxB,
           rowsA, rowsB, aggS, semI, semG):
    c = lax.axis_index("c")
    s = lax.axis_index("s")
    lo = c * dh
    base = s * epw
    sidx = (sidxA, sidxB)
    didx = (didxA, didxB)
    rows = (rowsA, rowsB)

    zv = jnp.zeros((16,), jnp.float32)

    def zrow(r, carry):
      for g in range(H // 16):
        rowsA[r, pl.ds(g * 16, 16)] = zv
      return carry

    lax.fori_loop(0, CHUNK, zrow, 0)

    def zcopy(k, carry):
      off = jnp.minimum(s * rps + k * CHUNK, srows - CHUNK)
      pltpu.sync_copy(rowsA, aggS.at[pl.ds(off, CHUNK)])
      return carry

    lax.fori_loop(0, zc, zcopy, 0)
    plsc.subcore_barrier()

    def icopy(t, b):
      off = base + t * CHUNK
      pltpu.async_copy(src_ref.at[pl.ds(off, CHUNK)], sidx[b], semI)
      pltpu.async_copy(dst_ref.at[pl.ds(off, CHUNK)], didx[b].at[0], semI)

    def iwait(b):
      pltpu.make_async_copy(src_ref.at[pl.ds(0, CHUNK)], sidx[b],
                            semI).wait()
      pltpu.make_async_copy(dst_ref.at[pl.ds(0, CHUNK)], didx[b].at[0],
                            semI).wait()

    def gather(b):
      pltpu.async_copy(table_ref.at[sidx[b]], rows[b], semG)

    def gwait(b):
      pltpu.make_async_copy(table_ref.at[sidxA], rows[b], semG).wait()

    def dcompute(b):
      for g in range(CHUNK // 16):
        d = didx[b][0, pl.ds(g * 16, 16)]
        li = d - lo
        valid = (d >= lo) & (li < dh)
        didx[b][0, pl.ds(g * 16, 16)] = jnp.where(valid, li, dh)

    def scatter(b):
      pltpu.sync_copy(rows[b], aggS.at[didx[b].at[0]], add=True)

    icopy(0, 0)
    iwait(0)
    icopy(1, 1)
    dcompute(0)
    gather(0)

    def chunk_step(t, b):
      gwait(b)

      @pl.when(t + 1 < TCH)
      def _():
        iwait(1 - b)
        gather(1 - b)

      scatter(b)

      @pl.when(t + 2 < TCH)
      def _():
        icopy(t + 2, b)

      @pl.when(t + 1 < TCH)
      def _():
        dcompute(1 - b)

    def pair(tt, carry):
      chunk_step(2 * tt, 0)
      chunk_step(2 * tt + 1, 1)
      return carry

    lax.fori_loop(0, TCH // 2, pair, 0)
    plsc.subcore_barrier()

    start = jnp.minimum(s * wps, dh - wps)
    pltpu.sync_copy(aggS.at[pl.ds(start, wps)],
                    agg_ref.at[pl.ds(lo + start, wps)])

  fn = pl.kernel(
      body,
      out_type=jax.ShapeDtypeStruct((n_dst, H), jnp.float32),
      mesh=plsc.VectorSubcoreMesh(core_axis_name="c", subcore_axis_name="s"),
      scratch_types=(
          pltpu.VMEM((CHUNK,), jnp.int32),
          pltpu.VMEM((CHUNK,), jnp.int32),
          pltpu.VMEM((1, CHUNK), jnp.int32),
          pltpu.VMEM((1, CHUNK), jnp.int32),
          pltpu.VMEM((CHUNK, H), jnp.float32),
          pltpu.VMEM((CHUNK, H), jnp.float32),
          pltpu.VMEM_SHARED((srows, H), jnp.float32),
          pltpu.SemaphoreType.DMA,
          pltpu.SemaphoreType.DMA,
      ),
  )
  return fn(table, src, dst)


def _edge_counts(d0, d1, d2, d3):
  dsts = (N_DX, N_ENC, N_MED, N_ENC)
  srows_max = max(_geom(n)[1] for n in dsts)

  def body(d0_ref, d1_ref, d2_ref, d3_ref, c0_ref, c1_ref, c2_ref, c3_ref,
           dstv, didx, onesb, zerob, cntS):
    c = lax.axis_index("c")
    s = lax.axis_index("s")

    zv = jnp.zeros((16,), jnp.float32)
    ov = jnp.ones((16,), jnp.float32)

    def frow(r, carry):
      for g in range(H // 16):
        zerob[r, pl.ds(g * 16, 16)] = zv
        onesb[r, pl.ds(g * 16, 16)] = ov
      return carry

    lax.fori_loop(0, CHUNK, frow, 0)

    for dst_ref, cnt_ref, n_dst in zip(
        (d0_ref, d1_ref, d2_ref, d3_ref),
        (c0_ref, c1_ref, c2_ref, c3_ref), dsts):
      dh, srows, rps, zc, wps = _geom(n_dst)
      lo = c * dh

      def zcopy(k, carry):
        off = jnp.minimum(s * rps + k * CHUNK, srows - CHUNK)
        pltpu.sync_copy(zerob, cntS.at[pl.ds(off, CHUNK)])
        return carry

      lax.fori_loop(0, zc, zcopy, 0)
      plsc.subcore_barrier()

      def step(t, carry):
        off = (s * TCH + t) * CHUNK
        pltpu.sync_copy(dst_ref.at[pl.ds(off, CHUNK)], dstv)
        for g in range(CHUNK // 16):
          d = dstv[pl.ds(g * 16, 16)]
          li = d - lo
          valid = (d >= lo) & (li < dh)
          didx[0, pl.ds(g * 16, 16)] = jnp.where(valid, li, dh)
        pltpu.sync_copy(onesb, cntS.at[didx.at[0]], add=True)
        return carry

      lax.fori_loop(0, TCH, step, 0)
      plsc.subcore_barrier()

      start = jnp.minimum(s * wps, dh - wps)
      pltpu.sync_copy(cntS.at[pl.ds(start, wps)],
                      cnt_ref.at[pl.ds(lo + start, wps)])
      plsc.subcore_barrier()

  fn = pl.kernel(
      body,
      out_type=tuple(
          jax.ShapeDtypeStruct((n, H), jnp.float32) for n in dsts),
      mesh=plsc.VectorSubcoreMesh(core_axis_name="c", subcore_axis_name="s"),
      scratch_types=(
          pltpu.VMEM((CHUNK,), jnp.int32),
          pltpu.VMEM((1, CHUNK), jnp.int32),
          pltpu.VMEM((CHUNK, H), jnp.float32),
          pltpu.VMEM((CHUNK, H), jnp.float32),
          pltpu.VMEM_SHARED((srows_max, H), jnp.float32),
      ),
  )
  return fn(d0, d1, d2, d3)


def _pad_edges(e):
  src = e[0].astype(jnp.int32)
  dst = e[1].astype(jnp.int32)
  pad = EP - E
  src = jnp.concatenate([src, jnp.zeros((pad,), jnp.int32)])
  dst = jnp.concatenate([dst, jnp.full((pad,), DST_PAD, jnp.int32)])
  return src, dst


def kernel(x_encounter, x_diagnosis, x_medication, edge_has_dx, edge_dx_of,
           edge_has_med, edge_med_of, W_enc, b_enc, Emb_dx, Emb_med, W_rel1,
           W_root1, b1, W_rel2, W_root2, b2, W_cls, b_cls):
  zb = jnp.zeros((H,), jnp.float32)

  h0_enc = _mm(x_encounter, W_enc, b_enc)
  h0_dx = _gather_rows(Emb_dx, x_diagnosis.astype(jnp.int32), N_DX)
  h0_med = _gather_rows(Emb_med, x_medication.astype(jnp.int32), N_MED)

  s0, d0 = _pad_edges(edge_has_dx)
  s1, d1 = _pad_edges(edge_dx_of)
  s2, d2 = _pad_edges(edge_has_med)
  s3, d3 = _pad_edges(edge_med_of)

  c0, c1, c2, c3 = _edge_counts(d0, d1, d2, d3)

  m0 = _mm(h0_enc, W_rel1[0], zb)
  m1 = _mm(h0_dx, W_rel1[1], zb)
  m2 = _mm(h0_enc, W_rel1[2], zb)
  m3 = _mm(h0_med, W_rel1[3], zb)
  r_enc = _mm(h0_enc, W_root1, b1)
  r_dx = _mm(h0_dx, W_root1, b1)
  r_med = _mm(h0_med, W_root1, b1)

  a0 = _seg_agg(m0, s0, d0, N_DX)
  a1 = _seg_agg(m1, s1, d1, N_ENC)
  a2 = _seg_agg(m2, s2, d2, N_MED)
  a3 = _seg_agg(m3, s3, d3, N_ENC)

  h1_dx = _comb1(r_dx, a0, c0)
  h1_med = _comb1(r_med, a2, c2)
  h1_enc = _comb2(r_enc, a1, c1, a3, c3)

  m1b = _mm(h1_dx, W_rel2[1], zb)
  m3b = _mm(h1_med, W_rel2[3], zb)
  r2 = _mm(h1_enc, W_root2, b2)

  a1b = _seg_agg(m1b, s1, d1, N_ENC)
  a3b = _seg_agg(m3b, s3, d3, N_ENC)

  logits = _final(r2, a1b, c1, a3b, c3, W_cls, b_cls)
  return logits.reshape(-1)
